# baseline (speedup 1.0000x reference)
import functools

import jax
import jax.numpy as jnp
from jax import lax
from jax.experimental import pallas as pl
from jax.experimental.pallas import tpu as pltpu
from jax.experimental.pallas import tpu_sc as plsc

N_USERS = 1000000
N_ITEMS = 100000
HIDDEN = 64
BATCH = 16384
FEAT_BITS = 17

NC = 2
NS = 16
NW = NC * NS
BPW = BATCH // NW
CHUNK = 128
NCHUNK = BPW // CHUNK
NBUF = 2
GROUP = 16
HALF = 256
NGROUP = HALF // GROUP

U1ROWS = (N_USERS + 127) // 128
I1ROWS = (N_ITEMS + 127) // 128

ROWW = 256


def _sc_gather(W2uf, W2if, W1up, W1ip, uidx, iidx, uw, iw):
    mesh = plsc.VectorSubcoreMesh(core_axis_name="c", subcore_axis_name="s")
    out_type = (
        jax.ShapeDtypeStruct((BATCH * ROWW,), jnp.float32),
        jax.ShapeDtypeStruct((BATCH, 128), jnp.float32),
        jax.ShapeDtypeStruct((BATCH, 128), jnp.float32),
    )

    @functools.partial(
        pl.kernel,
        mesh=mesh,
        out_type=out_type,
        scratch_types=[
            pltpu.VMEM((BPW,), jnp.int32),
            pltpu.VMEM((BPW,), jnp.int32),
            pltpu.VMEM((BPW,), jnp.int32),
            pltpu.VMEM((BPW,), jnp.int32),
            pltpu.VMEM((HALF * ROWW,), jnp.float32),
            pltpu.VMEM((CHUNK, 128), jnp.float32),
            pltpu.VMEM((CHUNK, 128), jnp.float32),
            pltpu.SemaphoreType.DMA,
            pltpu.SemaphoreType.DMA,
            pltpu.SemaphoreType.DMA,
            pltpu.SemaphoreType.DMA,
            pltpu.SemaphoreType.DMA,
        ],
    )
    def k(w2u_hbm, w2i_hbm, w1u_hbm, w1i_hbm, ui_hbm, ii_hbm, uw_hbm, iw_hbm,
          rows_hbm, g1u_hbm, g1i_hbm,
          ui_v, ii_v, uw_v, iw_v, stage, wb0, wb1,
          sem_r, g0, g1, w0, w1):
        wid = lax.axis_index("s") * NC + lax.axis_index("c")
        base = wid * BPW
        pltpu.sync_copy(ui_hbm.at[pl.ds(base, BPW)], ui_v)
        pltpu.sync_copy(ii_hbm.at[pl.ds(base, BPW)], ii_v)
        pltpu.sync_copy(uw_hbm.at[pl.ds(base, BPW)], uw_v)
        pltpu.sync_copy(iw_hbm.at[pl.ds(base, BPW)], iw_v)

        bufs = (wb0, wb1)
        gsems = (g0, g1)
        wsems = (w0, w1)
        streams = ((w1u_hbm, uw_v, g1u_hbm), (w1i_hbm, iw_v, g1i_hbm))
        descs = [(streams[t], c) for c in range(NCHUNK) for t in range(2)]
        nd = len(descs)

        def fire_gather(kk, b):
            (tbl, idxr, _), c = descs[kk]
            return pltpu.async_copy(
                tbl.at[idxr.at[pl.ds(c * CHUNK, CHUNK)]], bufs[b], gsems[b])

        def fire_write(kk, b):
            (_, _, outr), c = descs[kk]
            return pltpu.async_copy(
                bufs[b], outr.at[pl.ds(base + c * CHUNK, CHUNK)], wsems[b])

        gc = [fire_gather(0, 0), fire_gather(1, 1)]
        wc = [None, None]
        w1_step = [0]

        def w1_pipeline_step():
            kk = w1_step[0]
            if kk >= nd:
                return
            w1_step[0] = kk + 1
            b = kk % NBUF
            gc[b].wait()
            wc[b] = fire_write(kk, b)
            if kk + NBUF < nd:
                wc[b].wait()
                gc[b] = fire_gather(kk + NBUF, b)

        def fire_row_group(h, g):
            lo = h * HALF + g * GROUP

            @pl.loop(lo, lo + GROUP)
            def _(i):
                dst = (i - h * HALF) * ROWW
                u = ui_v[pl.ds(i, 1)][0]
                pltpu.async_copy(
                    w2u_hbm.at[pl.ds((u >> 1) * 128, 128)],
                    stage.at[pl.ds(dst, 128)], sem_r)
                v = ii_v[pl.ds(i, 1)][0]
                pltpu.async_copy(
                    w2i_hbm.at[pl.ds((v >> 1) * 128, 128)],
                    stage.at[pl.ds(dst + 128, 128)], sem_r)

        def drain_row_group():
            pltpu.make_async_copy(
                rows_hbm.at[pl.ds(0, GROUP * ROWW)],
                stage.at[pl.ds(0, GROUP * ROWW)], sem_r).wait()

        for h in range(BPW // HALF):
            for g in range(NGROUP):
                fire_row_group(h, g)
                if g >= 1:
                    drain_row_group()
                w1_pipeline_step()
            drain_row_group()
            pltpu.sync_copy(
                stage, rows_hbm.at[pl.ds((base + h * HALF) * ROWW,
                                         HALF * ROWW)])
        wc[0].wait()
        wc[1].wait()

    return k(W2uf, W2if, W1up, W1ip, uidx, iidx, uw, iw)


def _tc_body(ui_ref, ii_ref, f0_ref, f1_ref, rows_ref, g1u_ref, g1i_ref,
             w2f0_ref, w2f1_ref, w1f_ref, bias_ref, out_ref):
    j = lax.broadcasted_iota(jnp.int32, (1, 128), 1)
    mask = jnp.where(j < FEAT_BITS,
                     jnp.left_shift(1, jnp.maximum(FEAT_BITS - 1 - j, 0)), 0)
    bits0 = (jnp.bitwise_and(f0_ref[...], mask) != 0).astype(jnp.float32)
    bits1 = (jnp.bitwise_and(f1_ref[...], mask) != 0).astype(jnp.float32)
    s0 = jnp.sum(bits0, axis=1, keepdims=True)
    s1 = jnp.sum(bits1, axis=1, keepdims=True)

    w1f = w1f_ref[...]
    fo0 = jnp.sum(bits0 * w1f[0:1, :], axis=1, keepdims=True) / s0
    fo1 = jnp.sum(bits1 * w1f[1:2, :], axis=1, keepdims=True) / s1

    e0 = jnp.dot(bits0, w2f0_ref[...],
                 preferred_element_type=jnp.float32,
                 precision=lax.Precision.HIGHEST) / s0
    e1 = jnp.dot(bits1, w2f1_ref[...],
                 preferred_element_type=jnp.float32,
                 precision=lax.Precision.HIGHEST) / s1

    ui = ui_ref[...]
    ii = ii_ref[...]

    w1u = jnp.sum(g1u_ref[...] * (jnp.bitwise_and(ui, 127) == j),
                  axis=1, keepdims=True)
    w1i = jnp.sum(g1i_ref[...] * (jnp.bitwise_and(ii, 127) == j),
                  axis=1, keepdims=True)

    rows = rows_ref[...]
    uodd = jnp.bitwise_and(ui, 1) == 1
    iodd = jnp.bitwise_and(ii, 1) == 1
    u2 = jnp.where(uodd, rows[:, 64:128], rows[:, 0:64])
    i2 = jnp.where(iodd, rows[:, 192:256], rows[:, 128:192])
    ssum = u2 + i2 + e0 + e1
    diff = ssum * ssum - (u2 * u2 + i2 * i2 + e0 * e0 + e1 * e1)
    second = 0.5 * jnp.sum(diff, axis=1, keepdims=True)

    out_ref[...] = bias_ref[0, 0] + w1u + w1i + fo0 + fo1 + second


BB = 2048


def _tc_combine(ui, ii, f0, f1, rows, g1u, g1i, W2f0p, W2f1p, w1f, bias2):
    grid = (BATCH // BB,)
    bspec = lambda bs: pl.BlockSpec(bs, lambda i: (i, 0))
    wspec = lambda bs: pl.BlockSpec(bs, lambda i: (0, 0))
    return pl.pallas_call(
        _tc_body,
        grid=grid,
        in_specs=[
            bspec((BB, 1)), bspec((BB, 1)), bspec((BB, 1)), bspec((BB, 1)),
            bspec((BB, ROWW)),
            bspec((BB, 128)), bspec((BB, 128)),
            wspec((128, HIDDEN)), wspec((128, HIDDEN)),
            wspec((2, 128)), wspec((1, 1)),
        ],
        out_specs=bspec((BB, 1)),
        out_shape=jax.ShapeDtypeStruct((BATCH, 1), jnp.float32),
    )(ui, ii, f0, f1, rows, g1u, g1i, W2f0p, W2f1p, w1f, bias2)


def kernel(x, bias, W1u, W1i, W1f0, W1f1, W2u, W2i, W2f0, W2f1):
    uidx = x[:, 0]
    iidx = x[:, 1]
    uw = uidx // 128
    iw = iidx // 128

    W1up = jnp.concatenate(
        [W1u.reshape(-1), jnp.zeros((U1ROWS * 128 - N_USERS,), jnp.float32)]
    ).reshape(U1ROWS, 128)
    W1ip = jnp.concatenate(
        [W1i.reshape(-1), jnp.zeros((I1ROWS * 128 - N_ITEMS,), jnp.float32)]
    ).reshape(I1ROWS, 128)

    rows, g1u, g1i = _sc_gather(
        W2u.reshape(-1), W2i.reshape(-1), W1up, W1ip, uidx, iidx, uw, iw)

    pad = jnp.zeros((128 - FEAT_BITS, HIDDEN), jnp.float32)
    W2f0p = jnp.concatenate([W2f0, pad], axis=0)
    W2f1p = jnp.concatenate([W2f1, pad], axis=0)
    wpad = jnp.zeros((1, 128 - FEAT_BITS), jnp.float32)
    w1f = jnp.concatenate([
        jnp.concatenate([W1f0.T, wpad], axis=1),
        jnp.concatenate([W1f1.T, wpad], axis=1),
    ], axis=0)

    out = _tc_combine(
        x[:, 0:1], x[:, 1:2], x[:, 2:3], x[:, 3:4],
        rows.reshape(BATCH, ROWW), g1u, g1i,
        W2f0p, W2f1p, w1f, bias.reshape(1, 1),
    )
    return out[:, 0]
